# baseline (speedup 1.0000x reference)
import jax
import jax.numpy as jnp
from jax import lax
from jax.experimental import pallas as pl
from jax.experimental.pallas import tpu as pltpu
from jax.experimental.pallas import tpu_sc as plsc

_NC = 2
_NS = 16
_NW = _NC * _NS
_L = 16

_B = 4096
_S = 200
_D = 64
_C = 2
_BPW = _B // _NW
_CHUNK = _S // 2
_PAIRS = _BPW // 2


def _tec_body(table, idx_hbm, w_hbm, b_hbm, out_hbm,
              idx_v, buf_a, buf_b, w_v, b_v, log_v, sem_a, sem_b):
    wid = lax.axis_index("s") * _NC + lax.axis_index("c")
    base = wid * _BPW

    pltpu.sync_copy(idx_hbm.at[pl.ds(base, _BPW)], idx_v)
    pltpu.sync_copy(w_hbm, w_v)
    pltpu.sync_copy(b_hbm, b_v)

    inv_s = 1.0 / _S
    w = [[w_v[pl.ds(c * _D + k * _L, _L)] * inv_s for k in range(4)]
         for c in range(_C)]
    bvec = b_v[...]
    b0 = bvec[0]
    b1 = bvec[1]
    lane = jax.lax.iota(jnp.int32, 16)

    def issue(row, buf, sem):
        pltpu.async_copy(table.at[idx_v.at[row, 0]], buf.at[pl.ds(0, _CHUNK)], sem)
        pltpu.async_copy(table.at[idx_v.at[row, 1]], buf.at[pl.ds(_CHUNK, _CHUNK)], sem)

    def wait(buf, sem):
        pltpu.make_async_copy(table.at[pl.ds(0, _S)], buf, sem).wait()

    def process(row, buf, lvec):
        zero = jnp.zeros((_L,), jnp.float32)

        def acc_body(r, accs):
            a0, a1, a2, a3 = accs
            a0 = a0 + buf[r, pl.ds(0, _L)]
            a1 = a1 + buf[r, pl.ds(_L, _L)]
            a2 = a2 + buf[r, pl.ds(2 * _L, _L)]
            a3 = a3 + buf[r, pl.ds(3 * _L, _L)]
            return (a0, a1, a2, a3)

        a = lax.fori_loop(0, _S, acc_body, (zero, zero, zero, zero))
        t0 = a[0] * w[0][0] + a[1] * w[0][1] + a[2] * w[0][2] + a[3] * w[0][3]
        t1 = a[0] * w[1][0] + a[1] * w[1][1] + a[2] * w[1][2] + a[3] * w[1][3]
        l0 = jnp.sum(t0) + b0
        l1 = jnp.sum(t1) + b1
        slot = 2 * lax.rem(row, 8)
        lvec = jnp.where(lane == slot, l0, lvec)
        lvec = jnp.where(lane == slot + 1, l1, lvec)
        return lvec

    issue(0, buf_a, sem_a)

    def outer(i, lvec):
        issue(2 * i + 1, buf_b, sem_b)
        wait(buf_a, sem_a)
        lvec = process(2 * i, buf_a, lvec)

        @pl.when(i < _PAIRS - 1)
        def _():
            issue(2 * i + 2, buf_a, sem_a)

        wait(buf_b, sem_b)
        lvec = process(2 * i + 1, buf_b, lvec)

        @pl.when(lax.rem(i, 4) == 3)
        def _():
            log_v[pl.ds((i // 4) * _L, _L)] = lvec

        return lvec

    lax.fori_loop(0, _PAIRS, outer, jnp.zeros((_L,), jnp.float32))
    pltpu.sync_copy(log_v, out_hbm.at[pl.ds(base * _C, _BPW * _C)])


@jax.jit
def _sc_call(table, idx3, w_flat, b_pad):
    mesh = plsc.VectorSubcoreMesh(core_axis_name="c", subcore_axis_name="s",
                                  num_cores=_NC, num_subcores=_NS)
    return pl.kernel(
        _tec_body,
        out_type=jax.ShapeDtypeStruct((_B * _C,), jnp.float32),
        mesh=mesh,
        compiler_params=pltpu.CompilerParams(needs_layout_passes=False,
                                             use_tc_tiling_on_sc=False),
        scratch_types=[
            pltpu.VMEM((_BPW, 2, _CHUNK), jnp.int32),
            pltpu.VMEM((_S, _D), jnp.float32),
            pltpu.VMEM((_S, _D), jnp.float32),
            pltpu.VMEM((_C * _D,), jnp.float32),
            pltpu.VMEM((_L,), jnp.float32),
            pltpu.VMEM((_BPW * _C,), jnp.float32),
            pltpu.SemaphoreType.DMA,
            pltpu.SemaphoreType.DMA,
        ],
    )(table, idx3, w_flat, b_pad)


def kernel(x, embedding, W, b):
    idx3 = x.astype(jnp.int32).reshape(_B, 2, _CHUNK)
    w_flat = W.astype(jnp.float32).reshape(-1)
    b_pad = jnp.pad(b.astype(jnp.float32), (0, _L - _C))
    return _sc_call(embedding, idx3, w_flat, b_pad).reshape(_B, _C)
